# baseline (speedup 1.0000x reference)
import functools

import jax
import jax.numpy as jnp
from jax import lax
from jax.experimental import pallas as pl
from jax.experimental.pallas import tpu as pltpu
from jax.experimental.pallas import tpu_sc as plsc

_NC = 2
_NS = 16
_NW = _NC * _NS

_LANE = 128
_SUB = 8
_G = _LANE * _SUB
_NBUF = 2


@functools.cache
def _build_gather(B, D):
  assert B % (_NW * _G) == 0
  n_steps = B // (_NW * _G)
  n_chunks = B // (_NW * _LANE)
  b_per_w = B // _NW

  mesh = plsc.VectorSubcoreMesh(core_axis_name="c", subcore_axis_name="s")

  @functools.partial(
      pl.kernel,
      mesh=mesh,
      out_type=jax.ShapeDtypeStruct((B, D), jnp.float32),
      scratch_types=[
          pltpu.VMEM((n_chunks, _LANE), jnp.int32),
          pltpu.VMEM((_NBUF, _G, D), jnp.float32),
          pltpu.SemaphoreType.DMA((_NBUF,)),
      ],
      compiler_params=pltpu.CompilerParams(use_tc_tiling_on_sc=False),
  )
  def gather_kernel(idx_hbm, table_hbm, out_hbm, idx_v, rows_v, gsems):
    wid = lax.axis_index("s") * _NC + lax.axis_index("c")
    base = wid * b_per_w
    pltpu.sync_copy(idx_hbm.at[wid], idx_v)

    def issue(g, b):
      for j in range(_SUB):
        pltpu.async_copy(
            table_hbm.at[idx_v.at[g * _SUB + j]],
            rows_v.at[b, pl.ds(j * _LANE, _LANE)],
            gsems.at[b],
        )

    for b in range(_NBUF):
      issue(b, b)

    def step(g, carry):
      b = lax.rem(g, _NBUF)
      pltpu.make_async_copy(
          table_hbm.at[pl.ds(0, _G)], rows_v.at[b], gsems.at[b]).wait()
      pltpu.sync_copy(rows_v.at[b], out_hbm.at[pl.ds(base + g * _G, _G)])

      @pl.when(g + _NBUF < n_steps)
      def _():
        issue(g + _NBUF, b)

      return carry

    lax.fori_loop(0, n_steps, step, 0)

  return gather_kernel


_TBLK = 1024


@functools.cache
def _build_repack(V, D):
  pack = 128 // D

  def body(x_ref, o_ref):
    x = x_ref[...]
    y = x.T.reshape(_TBLK // pack, pack, D)
    o_ref[...] = jnp.concatenate([y[:, j, :] for j in range(pack)], axis=1)

  grid = (V + _TBLK - 1) // _TBLK
  return pl.pallas_call(
      body,
      grid=(grid,),
      in_specs=[pl.BlockSpec((D, _TBLK), lambda i: (0, i))],
      out_specs=pl.BlockSpec((_TBLK // pack, 128), lambda i: (i, 0)),
      out_shape=jax.ShapeDtypeStruct((V // pack, 128), jnp.float32),
  )


def kernel(inputs, table):
  bt, s, n = inputs.shape
  b = bt * s * n
  v, d = table.shape
  idx = inputs.reshape(_NW, b // (_NW * _LANE), _LANE).astype(jnp.int32)
  packed = _build_repack(v, d)(table.T)
  table_lin = packed.reshape(v, d)
  out = _build_gather(b, d)(idx, table_lin)
  return out.reshape(bt, s, n, d)


# speedup vs baseline: 1.6148x; 1.6148x over previous
import functools

import jax
import jax.numpy as jnp
from jax import lax
from jax.experimental import pallas as pl
from jax.experimental.pallas import tpu as pltpu
from jax.experimental.pallas import tpu_sc as plsc

_NC = 2
_NS = 16
_NW = _NC * _NS

_LANE = 128
_SUB = 8
_G = _LANE * _SUB
_NBUF = 2

_TBLK = 2048


@functools.cache
def _build_repack(V, D):
  assert D == 32
  n_grp = _TBLK // 512
  vpad = ((V + 511) // 512) * 512

  def body(x_ref, o_ref):
    x = x_ref[...]
    for q in range(n_grp):
      xq = x[:, q * 512:(q + 1) * 512]
      stacked = jnp.concatenate(
          [xq[:, c * 128:(c + 1) * 128] for c in range(4)], axis=0)
      o_ref[q * 128:(q + 1) * 128, :] = stacked.T

  grid = (V + _TBLK - 1) // _TBLK
  return pl.pallas_call(
      body,
      grid=(grid,),
      in_specs=[pl.BlockSpec((D, _TBLK), lambda i: (0, i))],
      out_specs=pl.BlockSpec((_TBLK // 4, 128), lambda i: (i, 0)),
      out_shape=jax.ShapeDtypeStruct((vpad // 4, 128), jnp.float32),
  )


@functools.cache
def _build_gather(B, D, VS):
  assert B % (_NW * _G) == 0
  n_steps = B // (_NW * _G)
  n_chunks = B // (_NW * _LANE)
  b_per_w = B // _NW

  mesh = plsc.VectorSubcoreMesh(core_axis_name="c", subcore_axis_name="s")

  @functools.partial(
      pl.kernel,
      mesh=mesh,
      out_type=jax.ShapeDtypeStruct((B, D), jnp.float32),
      scratch_types=[
          pltpu.VMEM((n_chunks, _LANE), jnp.int32),
          pltpu.VMEM((_NBUF, _G, D), jnp.float32),
          pltpu.SemaphoreType.DMA((_NBUF,)),
      ],
      compiler_params=pltpu.CompilerParams(use_tc_tiling_on_sc=False),
  )
  def gather_kernel(idx_hbm, table_hbm, out_hbm, idx_v, rows_v, gsems):
    wid = lax.axis_index("s") * _NC + lax.axis_index("c")
    base = wid * b_per_w
    pltpu.sync_copy(idx_hbm.at[wid], idx_v)

    def remap(c, carry):
      for q in range(_LANE // 16):
        v = idx_v[c, pl.ds(q * 16, 16)]
        g = ((v & jnp.int32(~511)) | ((v & jnp.int32(127)) << 2)
             | ((v >> 7) & jnp.int32(3)))
        idx_v[c, pl.ds(q * 16, 16)] = g
      return carry

    lax.fori_loop(0, n_chunks, remap, 0)

    def issue(g, b):
      for j in range(_SUB):
        pltpu.async_copy(
            table_hbm.at[idx_v.at[g * _SUB + j]],
            rows_v.at[b, pl.ds(j * _LANE, _LANE)],
            gsems.at[b],
        )

    for b in range(_NBUF):
      issue(b, b)

    def step(g, carry):
      b = lax.rem(g, _NBUF)
      pltpu.make_async_copy(
          table_hbm.at[pl.ds(0, _G)], rows_v.at[b], gsems.at[b]).wait()
      pltpu.sync_copy(rows_v.at[b], out_hbm.at[pl.ds(base + g * _G, _G)])

      @pl.when(g + _NBUF < n_steps)
      def _():
        issue(g + _NBUF, b)

      return carry

    lax.fori_loop(0, n_steps, step, 0)

  return gather_kernel


def kernel(inputs, table):
  bt, s, n = inputs.shape
  b = bt * s * n
  v, d = table.shape
  idx = inputs.reshape(_NW, b // (_NW * _LANE), _LANE).astype(jnp.int32)
  packed = _build_repack(v, d)(table.T)
  vs = packed.shape[0] * packed.shape[1] // d
  table_lin = packed.reshape(vs, d)
  out = _build_gather(b, d, vs)(idx, table_lin)
  return out.reshape(bt, s, n, d)


# speedup vs baseline: 2.2681x; 1.4046x over previous
import functools

import jax
import jax.numpy as jnp
from jax import lax
from jax.experimental import pallas as pl
from jax.experimental.pallas import tpu as pltpu
from jax.experimental.pallas import tpu_sc as plsc

_NC = 2
_NS = 16
_NW = _NC * _NS

_LANE = 128
_SUB = 8
_G = _LANE * _SUB
_NBUF = 2

_TBLK = 8192


@functools.cache
def _build_repack(V, D):
  assert D == 32
  n_grp = _TBLK // 512
  vpad = ((V + 511) // 512) * 512

  def body(x_ref, o_ref):
    x = x_ref[...]
    for q in range(n_grp):
      xq = x[:, q * 512:(q + 1) * 512]
      stacked = jnp.concatenate(
          [xq[:, c * 128:(c + 1) * 128] for c in range(4)], axis=0)
      o_ref[q * 128:(q + 1) * 128, :] = stacked.T

  grid = (V + _TBLK - 1) // _TBLK
  return pl.pallas_call(
      body,
      grid=(grid,),
      in_specs=[pl.BlockSpec((D, _TBLK), lambda i: (0, i))],
      out_specs=pl.BlockSpec((_TBLK // 4, 128), lambda i: (i, 0)),
      out_shape=jax.ShapeDtypeStruct((vpad // 4, 128), jnp.float32),
  )


@functools.cache
def _build_gather(B, D, VS):
  assert B % (_NW * _G) == 0
  n_steps = B // (_NW * _G)
  n_chunks = B // (_NW * _LANE)
  b_per_w = B // _NW

  mesh = plsc.VectorSubcoreMesh(core_axis_name="c", subcore_axis_name="s")

  @functools.partial(
      pl.kernel,
      mesh=mesh,
      out_type=jax.ShapeDtypeStruct((B, D), jnp.float32),
      scratch_types=[
          pltpu.VMEM((n_chunks, _LANE), jnp.int32),
          pltpu.VMEM((_NBUF, _G, D), jnp.float32),
          pltpu.SemaphoreType.DMA((_NBUF,)),
      ],
      compiler_params=pltpu.CompilerParams(use_tc_tiling_on_sc=False),
  )
  def gather_kernel(idx_hbm, table_hbm, out_hbm, idx_v, rows_v, gsems):
    wid = lax.axis_index("s") * _NC + lax.axis_index("c")
    base = wid * b_per_w
    pltpu.sync_copy(idx_hbm.at[wid], idx_v)

    def remap(c, carry):
      for q in range(_LANE // 16):
        v = idx_v[c, pl.ds(q * 16, 16)]
        g = ((v & jnp.int32(~511)) | ((v & jnp.int32(127)) << 2)
             | ((v >> 7) & jnp.int32(3)))
        idx_v[c, pl.ds(q * 16, 16)] = g
      return carry

    lax.fori_loop(0, n_chunks, remap, 0)

    def issue(g, b):
      for j in range(_SUB):
        pltpu.async_copy(
            table_hbm.at[idx_v.at[g * _SUB + j]],
            rows_v.at[b, pl.ds(j * _LANE, _LANE)],
            gsems.at[b],
        )

    for b in range(_NBUF):
      issue(b, b)

    def step(g, carry):
      b = lax.rem(g, _NBUF)
      pltpu.make_async_copy(
          table_hbm.at[pl.ds(0, _G)], rows_v.at[b], gsems.at[b]).wait()
      pltpu.sync_copy(rows_v.at[b], out_hbm.at[pl.ds(base + g * _G, _G)])

      @pl.when(g + _NBUF < n_steps)
      def _():
        issue(g + _NBUF, b)

      return carry

    lax.fori_loop(0, n_steps, step, 0)

  return gather_kernel


def kernel(inputs, table):
  bt, s, n = inputs.shape
  b = bt * s * n
  v, d = table.shape
  idx = inputs.reshape(_NW, b // (_NW * _LANE), _LANE).astype(jnp.int32)
  packed = _build_repack(v, d)(table.T)
  vs = packed.shape[0] * packed.shape[1] // d
  table_lin = packed.reshape(vs, d)
  out = _build_gather(b, d, vs)(idx, table_lin)
  return out.reshape(bt, s, n, d)


# speedup vs baseline: 2.5656x; 1.1312x over previous
import functools

import jax
import jax.numpy as jnp
from jax import lax
from jax.experimental import pallas as pl
from jax.experimental.pallas import tpu as pltpu
from jax.experimental.pallas import tpu_sc as plsc

_NC = 2
_NS = 16
_NW = _NC * _NS

_LANE = 128
_SUB = 8
_G = _LANE * _SUB
_NBUF = 2

_TBLK = 32768


@functools.cache
def _build_repack(V, D):
  assert D == 32
  n_grp = _TBLK // 512
  vpad = ((V + 511) // 512) * 512

  def body(x_ref, o_ref):
    x = x_ref[...]
    for q in range(n_grp):
      xq = x[:, q * 512:(q + 1) * 512]
      stacked = jnp.concatenate(
          [xq[:, c * 128:(c + 1) * 128] for c in range(4)], axis=0)
      o_ref[q * 128:(q + 1) * 128, :] = stacked.T

  grid = (V + _TBLK - 1) // _TBLK
  return pl.pallas_call(
      body,
      grid=(grid,),
      in_specs=[pl.BlockSpec((D, _TBLK), lambda i: (0, i))],
      out_specs=pl.BlockSpec((_TBLK // 4, 128), lambda i: (i, 0)),
      out_shape=jax.ShapeDtypeStruct((vpad // 4, 128), jnp.float32),
  )


@functools.cache
def _build_gather(B, D, VS):
  assert B % (_NW * _G) == 0
  n_steps = B // (_NW * _G)
  n_chunks = B // (_NW * _LANE)
  b_per_w = B // _NW

  mesh = plsc.VectorSubcoreMesh(core_axis_name="c", subcore_axis_name="s")

  @functools.partial(
      pl.kernel,
      mesh=mesh,
      out_type=jax.ShapeDtypeStruct((B, D), jnp.float32),
      scratch_types=[
          pltpu.VMEM((n_chunks, _LANE), jnp.int32),
          pltpu.VMEM((_NBUF, _G, D), jnp.float32),
          pltpu.SemaphoreType.DMA((_NBUF,)),
      ],
      compiler_params=pltpu.CompilerParams(use_tc_tiling_on_sc=False),
  )
  def gather_kernel(idx_hbm, table_hbm, out_hbm, idx_v, rows_v, gsems):
    wid = lax.axis_index("s") * _NC + lax.axis_index("c")
    base = wid * b_per_w
    pltpu.sync_copy(idx_hbm.at[wid], idx_v)

    def remap(c, carry):
      for q in range(_LANE // 16):
        v = idx_v[c, pl.ds(q * 16, 16)]
        g = ((v & jnp.int32(~511)) | ((v & jnp.int32(127)) << 2)
             | ((v >> 7) & jnp.int32(3)))
        idx_v[c, pl.ds(q * 16, 16)] = g
      return carry

    lax.fori_loop(0, n_chunks, remap, 0)

    def issue(g, b):
      for j in range(_SUB):
        pltpu.async_copy(
            table_hbm.at[idx_v.at[g * _SUB + j]],
            rows_v.at[b, pl.ds(j * _LANE, _LANE)],
            gsems.at[b],
        )

    for b in range(_NBUF):
      issue(b, b)

    def step(g, carry):
      b = lax.rem(g, _NBUF)
      pltpu.make_async_copy(
          table_hbm.at[pl.ds(0, _G)], rows_v.at[b], gsems.at[b]).wait()
      pltpu.sync_copy(rows_v.at[b], out_hbm.at[pl.ds(base + g * _G, _G)])

      @pl.when(g + _NBUF < n_steps)
      def _():
        issue(g + _NBUF, b)

      return carry

    lax.fori_loop(0, n_steps, step, 0)

  return gather_kernel


def kernel(inputs, table):
  bt, s, n = inputs.shape
  b = bt * s * n
  v, d = table.shape
  idx = inputs.reshape(_NW, b // (_NW * _LANE), _LANE).astype(jnp.int32)
  packed = _build_repack(v, d)(table.T)
  vs = packed.shape[0] * packed.shape[1] // d
  table_lin = packed.reshape(vs, d)
  out = _build_gather(b, d, vs)(idx, table_lin)
  return out.reshape(bt, s, n, d)


# speedup vs baseline: 2.5844x; 1.0073x over previous
import functools

import jax
import jax.numpy as jnp
from jax import lax
from jax.experimental import pallas as pl
from jax.experimental.pallas import tpu as pltpu
from jax.experimental.pallas import tpu_sc as plsc

_NC = 2
_NS = 16
_NW = _NC * _NS

_LANE = 128
_SUB = 8
_G = _LANE * _SUB
_NBUF = 3

_TBLK = 65536


@functools.cache
def _build_repack(V, D):
  assert D == 32
  n_grp = _TBLK // 512
  vpad = ((V + 511) // 512) * 512

  def body(x_ref, o_ref):
    x = x_ref[...]
    for q in range(n_grp):
      xq = x[:, q * 512:(q + 1) * 512]
      stacked = jnp.concatenate(
          [xq[:, c * 128:(c + 1) * 128] for c in range(4)], axis=0)
      o_ref[q * 128:(q + 1) * 128, :] = stacked.T

  grid = (V + _TBLK - 1) // _TBLK
  return pl.pallas_call(
      body,
      grid=(grid,),
      in_specs=[pl.BlockSpec((D, _TBLK), lambda i: (0, i))],
      out_specs=pl.BlockSpec((_TBLK // 4, 128), lambda i: (i, 0)),
      out_shape=jax.ShapeDtypeStruct((vpad // 4, 128), jnp.float32),
  )


@functools.cache
def _build_gather(B, D, VS):
  assert B % (_NW * _G) == 0
  n_steps = B // (_NW * _G)
  n_chunks = B // (_NW * _LANE)
  b_per_w = B // _NW

  mesh = plsc.VectorSubcoreMesh(core_axis_name="c", subcore_axis_name="s")

  @functools.partial(
      pl.kernel,
      mesh=mesh,
      out_type=jax.ShapeDtypeStruct((B, D), jnp.float32),
      scratch_types=[
          pltpu.VMEM((n_chunks, _LANE), jnp.int32),
          pltpu.VMEM((_NBUF, _G, D), jnp.float32),
          pltpu.SemaphoreType.DMA((_NBUF,)),
      ],
      compiler_params=pltpu.CompilerParams(use_tc_tiling_on_sc=False),
  )
  def gather_kernel(idx_hbm, table_hbm, out_hbm, idx_v, rows_v, gsems):
    wid = lax.axis_index("s") * _NC + lax.axis_index("c")
    base = wid * b_per_w
    pltpu.sync_copy(idx_hbm.at[wid], idx_v)

    def remap(c, carry):
      for q in range(_LANE // 16):
        v = idx_v[c, pl.ds(q * 16, 16)]
        g = ((v & jnp.int32(~511)) | ((v & jnp.int32(127)) << 2)
             | ((v >> 7) & jnp.int32(3)))
        idx_v[c, pl.ds(q * 16, 16)] = g
      return carry

    lax.fori_loop(0, n_chunks, remap, 0)

    def issue(g, b):
      for j in range(_SUB):
        pltpu.async_copy(
            table_hbm.at[idx_v.at[g * _SUB + j]],
            rows_v.at[b, pl.ds(j * _LANE, _LANE)],
            gsems.at[b],
        )

    for b in range(_NBUF):
      issue(b, b)

    def step(g, carry):
      b = lax.rem(g, _NBUF)
      pltpu.make_async_copy(
          table_hbm.at[pl.ds(0, _G)], rows_v.at[b], gsems.at[b]).wait()
      pltpu.sync_copy(rows_v.at[b], out_hbm.at[pl.ds(base + g * _G, _G)])

      @pl.when(g + _NBUF < n_steps)
      def _():
        issue(g + _NBUF, b)

      return carry

    lax.fori_loop(0, n_steps, step, 0)

  return gather_kernel


def kernel(inputs, table):
  bt, s, n = inputs.shape
  b = bt * s * n
  v, d = table.shape
  idx = inputs.reshape(_NW, b // (_NW * _LANE), _LANE).astype(jnp.int32)
  packed = _build_repack(v, d)(table.T)
  vs = packed.shape[0] * packed.shape[1] // d
  table_lin = packed.reshape(vs, d)
  out = _build_gather(b, d, vs)(idx, table_lin)
  return out.reshape(bt, s, n, d)
